# Initial kernel scaffold; baseline (speedup 1.0000x reference)
import jax
import jax.numpy as jnp
from jax.experimental import pallas as pl


def kernel(params, x, edge_index, edge_attr, batch):
    raise NotImplementedError("write your pallas kernel here")


# speedup vs baseline: 1.3990x; 1.3990x over previous
import functools
import jax
import jax.numpy as jnp
from jax import lax
from jax.experimental import pallas as pl
from jax.experimental.pallas import tpu as pltpu
from jax.experimental.pallas import tpu_sc as plsc

N = 10000
NE = 320000
HID = 128
DE = 16
L = 3
NG = 128
ODIM = 10

NW = 32
NPT = 320
NPAD = NW * NPT
W = 128
CH = 256
EPT = NE // NW
WE = 2000
ROWS_PER_TILE = NPAD // 16

_mesh = plsc.VectorSubcoreMesh(core_axis_name="c", subcore_axis_name="s")


def _wid():
    return lax.axis_index("s") * 2 + lax.axis_index("c")


CAP = NE

@functools.partial(
    pl.kernel,
    mesh=_mesh,
    out_type=[
        jax.ShapeDtypeStruct((NPAD, DE), jnp.float32),
        jax.ShapeDtypeStruct((NPAD, DE), jnp.float32),
        jax.ShapeDtypeStruct((NW, CAP), jnp.int32),
        jax.ShapeDtypeStruct((NW, CAP), jnp.int32),
        jax.ShapeDtypeStruct((NW, 16), jnp.int32),
    ],
    scratch_types=[
        pltpu.VMEM((CH,), jnp.int32),
        pltpu.VMEM((CH,), jnp.int32),
        pltpu.VMEM((CH, DE), jnp.float32),
        pltpu.VMEM((W + 32,), jnp.int32),
        pltpu.VMEM((W + 32,), jnp.int32),
        pltpu.VMEM((16,), jnp.int32),
        pltpu.VMEM((48,), jnp.int32),
        pltpu.VMEM((NPT + 1, DE), jnp.float32),
        pltpu.VMEM((NPT + 1, DE), jnp.float32),
    ],
)
def _bucket(src_hbm, dst_hbm, ea_hbm, eout, dout, slist, dlist, cnts,
            dvm, svm, eav, st_src, st_dst, cntb, red, acc_e, acc_d):
    wid = _wid()
    lo = wid * NPT
    iota16 = lax.iota(jnp.int32, 16)
    zv = jnp.zeros((16,), jnp.float32)
    ov = jnp.ones((16,), jnp.float32)
    zi = jnp.zeros((16,), jnp.int32)

    def init(i, _):
        acc_e[i, pl.ds(0, DE)] = zv
        acc_d[i, pl.ds(0, DE)] = zv
        return 0
    lax.fori_loop(0, NPT + 1, init, 0)

    def flush(wc):
        pltpu.sync_copy(st_src.at[pl.ds(0, W)], slist.at[wid, pl.ds(wc * W, W)])
        pltpu.sync_copy(st_dst.at[pl.ds(0, W)], dlist.at[wid, pl.ds(wc * W, W)])
        return wc + 1

    def chunk(ci, carry):
        off = ci * CH
        pltpu.sync_copy(dst_hbm.at[pl.ds(off, CH)], dvm)
        pltpu.sync_copy(src_hbm.at[pl.ds(off, CH)], svm)
        pltpu.sync_copy(ea_hbm.at[pl.ds(off, CH)], eav)

        def vreg(k, carry):
            n, wc = carry
            d16 = dvm[pl.ds(k * 16, 16)]
            dloc = d16 - lo
            m = (dloc >= 0) & (dloc < NPT)
            mi = jnp.where(m, 1, 0)
            red[pl.ds(16, 16)] = mi
            red[pl.ds(0, 16)] = zi
            x = red[pl.ds(16, 16)] + red[pl.ds(15, 16)]
            red[pl.ds(16, 16)] = x
            x = red[pl.ds(16, 16)] + red[pl.ds(14, 16)]
            red[pl.ds(16, 16)] = x
            x = red[pl.ds(16, 16)] + red[pl.ds(12, 16)]
            red[pl.ds(16, 16)] = x
            x = red[pl.ds(16, 16)] + red[pl.ds(8, 16)]
            tot = x[15]

            def work(carry):
                s16 = svm[pl.ds(k * 16, 16)]

                def lane(l, carry):
                    def app(carry):
                        n, wc = carry
                        dl = dloc[l]
                        st_src[pl.ds(n, 16)] = jnp.full((16,), s16[l], jnp.int32)
                        st_dst[pl.ds(n, 16)] = jnp.full((16,), dl, jnp.int32)
                        acc_e[dl, pl.ds(0, DE)] = acc_e[dl, pl.ds(0, DE)] + eav[k * 16 + l, pl.ds(0, DE)]
                        acc_d[dl, pl.ds(0, DE)] = acc_d[dl, pl.ds(0, DE)] + ov
                        n = n + 1

                        def fl(carry):
                            n, wc = carry
                            wc = flush(wc)
                            st_src[pl.ds(0, 16)] = st_src[pl.ds(W, 16)]
                            st_dst[pl.ds(0, 16)] = st_dst[pl.ds(W, 16)]
                            return n - W, wc

                        return lax.cond(n >= W, fl, lambda c: c, (n, wc))

                    return lax.cond(mi[l] > 0, app, lambda c: c, carry)

                for l in range(16):
                    carry = lane(l, carry)
                return carry

            return lax.cond(tot > 0, work, lambda c: c, (n, wc))

        return lax.fori_loop(0, CH // 16, vreg, carry)

    n, wc = lax.fori_loop(0, NE // CH, chunk, (jnp.int32(0), jnp.int32(0)))

    def pad(k, _):
        vals_s = st_src[pl.ds(k * 16, 16)]
        vals_d = st_dst[pl.ds(k * 16, 16)]
        ok = (k * 16 + iota16) < n
        st_src[pl.ds(k * 16, 16)] = jnp.where(ok, vals_s, 0)
        st_dst[pl.ds(k * 16, 16)] = jnp.where(ok, vals_d, NPT)
        return 0
    lax.fori_loop(0, W // 16, pad, 0)
    wc = flush(wc)

    cntb[pl.ds(0, 16)] = jnp.full((16,), wc * W, jnp.int32)
    pltpu.sync_copy(cntb, cnts.at[wid])

    pltpu.sync_copy(acc_e.at[pl.ds(0, NPT)], eout.at[pl.ds(lo, NPT)])
    pltpu.sync_copy(acc_d.at[pl.ds(0, NPT)], dout.at[pl.ds(lo, NPT)])


@functools.partial(
    pl.kernel,
    mesh=_mesh,
    out_type=[
        jax.ShapeDtypeStruct((NPAD, HID), jnp.float32),
        jax.ShapeDtypeStruct((NPAD, HID), jnp.float32),
    ],
    scratch_types=[
        pltpu.VMEM((16,), jnp.int32),
        pltpu.VMEM((W,), jnp.int32),
        pltpu.VMEM((W + 16,), jnp.int32),
        pltpu.VMEM((W, HID), jnp.float32),
        pltpu.VMEM((NPT + 1, HID), jnp.float32),
        pltpu.VMEM((NPT + 1, HID), jnp.float32),
    ],
)
def _aggregate(slist, dlist, cnts, x_hbm, sum_out, max_out,
               cntv, gidx, dstw, rows_v, acc_s, acc_m):
    wid = _wid()
    lo = wid * NPT
    zv = jnp.zeros((16,), jnp.float32)
    nv = jnp.full((16,), -3.0e38, jnp.float32)

    def init(i, _):
        for j in range(8):
            acc_s[i, pl.ds(j * 16, 16)] = zv
            acc_m[i, pl.ds(j * 16, 16)] = nv
        return 0
    lax.fori_loop(0, NPT + 1, init, 0)

    pltpu.sync_copy(cnts.at[wid], cntv)
    nwin = cntv[pl.ds(0, 16)][0] // W

    def win(w, _):
        pltpu.sync_copy(slist.at[wid, pl.ds(w * W, W)], gidx)
        pltpu.sync_copy(dlist.at[wid, pl.ds(w * W, W)], dstw.at[pl.ds(0, W)])
        pltpu.sync_copy(x_hbm.at[gidx], rows_v)

        def edge(e, _):
            dl = dstw[pl.ds(e, 16)][0]
            for j in range(8):
                r = rows_v[e, pl.ds(j * 16, 16)]
                acc_s[dl, pl.ds(j * 16, 16)] = acc_s[dl, pl.ds(j * 16, 16)] + r
                acc_m[dl, pl.ds(j * 16, 16)] = jnp.maximum(acc_m[dl, pl.ds(j * 16, 16)], r)
            return 0
        lax.fori_loop(0, W, edge, 0)
        return 0

    lax.fori_loop(0, nwin, win, 0)

    pltpu.sync_copy(acc_s.at[pl.ds(0, NPT)], sum_out.at[pl.ds(lo, NPT)])
    pltpu.sync_copy(acc_m.at[pl.ds(0, NPT)], max_out.at[pl.ds(lo, NPT)])


@functools.partial(
    pl.kernel,
    mesh=_mesh,
    out_type=jax.ShapeDtypeStruct((2, NG, HID), jnp.float32),
    scratch_types=[
        pltpu.VMEM((NPT,), jnp.int32),
        pltpu.VMEM((NPT, HID), jnp.float32),
        pltpu.VMEM((8, HID), jnp.float32),
        pltpu.VMEM_SHARED((NG, HID), jnp.float32),
    ],
)
def _pool(batch_hbm, x_hbm, out, bidx, rows_v, z_v, shp):
    cid = lax.axis_index("c")
    sid = lax.axis_index("s")
    zv = jnp.zeros((16,), jnp.float32)

    def zfill(i, _):
        r = i // 8
        col = (i - r * 8) * 16
        z_v[r, pl.ds(col, 16)] = zv
        return 0
    lax.fori_loop(0, 8 * HID // 16, zfill, 0)
    pltpu.sync_copy(z_v, shp.at[pl.ds(sid * 8, 8)])
    plsc.subcore_barrier()

    lo = _wid() * NPT
    pltpu.sync_copy(batch_hbm.at[pl.ds(lo, NPT)], bidx)
    pltpu.sync_copy(x_hbm.at[pl.ds(lo, NPT)], rows_v)
    pltpu.sync_copy(rows_v, shp.at[bidx], add=True)
    plsc.subcore_barrier()

    @pl.when(sid == 0)
    def _():
        pltpu.sync_copy(shp, out.at[cid])


def _bn_elu(y, gamma, beta, mask, apply_elu):
    cnt = jnp.float32(N)
    ym = jnp.where(mask, y, 0.0)
    mean = jnp.sum(ym, axis=0, keepdims=True) / cnt
    d = y - mean
    var = jnp.sum(jnp.where(mask, d * d, 0.0), axis=0, keepdims=True) / cnt
    out = d / jnp.sqrt(var + 1e-5) * gamma + beta
    if apply_elu:
        out = jnp.where(out > 0, out, jnp.exp(jnp.minimum(out, 0.0)) - 1.0)
    return jnp.where(mask, out, 0.0)


def _rowmask():
    return lax.broadcasted_iota(jnp.int32, (NPAD, 1), 0) < N


def _tc1(e_in, d_in, g0_ref, vecs_ref, sf_ref, h1_ref, x2_ref):
    E = e_in[...]
    deg = d_in[:, 0:1]
    u = vecs_ref[0:1, :]
    v = vecs_ref[1:2, :]
    b0 = vecs_ref[2:3, :]
    h0 = vecs_ref[3:4, :]
    gam = vecs_ref[4:5, :]
    bet = vecs_ref[5:6, :]
    mask = _rowmask()
    h1 = deg * u + jnp.where(deg > 0, 1.0, 0.0) * v \
        + jnp.dot(E, g0_ref[...], preferred_element_type=jnp.float32) + b0
    se0 = sf_ref[0:1, 0:1]
    se1 = sf_ref[1:2, 0:1]
    f0 = sf_ref[2:3, 0:1]
    f1 = sf_ref[3:4, 0:1]
    f2 = sf_ref[4:5, 0:1]
    a = se0 * h0
    bmat = se1 * h1
    s = a + bmat
    y = f0 * s + f1 * (s * 0.5) + f2 * jnp.maximum(a, bmat)
    x2 = _bn_elu(y, gam, bet, mask, True)
    h1_ref[...] = h1
    x2_ref[...] = x2


def _tc_mid(nh, sum_ref, max_ref, e_ref, deg_ref, ws_ref, g_ref, vecs_ref, sf_ref,
            *rest):
    hs_refs = rest[:nh]
    h_ref, x2_ref = rest[nh], rest[nh + 1]
    deg = deg_ref[:, 0:1]
    invd = 1.0 / jnp.maximum(deg, 1.0)
    asum = sum_ref[...]
    amax = jnp.where(deg > 0, max_ref[...], 0.0)
    b_ = vecs_ref[0:1, :]
    h0 = vecs_ref[1:2, :]
    gam = vecs_ref[2:3, :]
    bet = vecs_ref[3:4, :]
    h = jnp.dot(asum, ws_ref[0], preferred_element_type=jnp.float32) \
        + jnp.dot(asum * invd, ws_ref[1], preferred_element_type=jnp.float32) \
        + jnp.dot(amax, ws_ref[2], preferred_element_type=jnp.float32) \
        + jnp.dot(e_ref[...], g_ref[...], preferred_element_type=jnp.float32) + b_
    hs = [h0] + [r[...] for r in hs_refs] + [h]
    k = len(hs)
    terms = [sf_ref[j:j + 1, 0:1] * hs[j] for j in range(k)]
    s = terms[0]
    mx = terms[0]
    for t in terms[1:]:
        s = s + t
        mx = jnp.maximum(mx, t)
    f0 = sf_ref[k:k + 1, 0:1]
    f1 = sf_ref[k + 1:k + 2, 0:1]
    f2 = sf_ref[k + 2:k + 3, 0:1]
    y = f0 * s + f1 * (s / jnp.float32(k)) + f2 * mx
    mask = _rowmask()
    x2 = _bn_elu(y, gam, bet, mask, nh < 2)
    h_ref[...] = h
    x2_ref[...] = x2


def _tc4(pp_ref, cw_ref, cb_ref, out_ref):
    pooled = pp_ref[0] + pp_ref[1]
    out_ref[...] = jnp.dot(pooled, cw_ref[...], preferred_element_type=jnp.float32) \
        + cb_ref[0:1, :]


def _call_tc(fn, out_shapes, *args):
    return pl.pallas_call(
        fn,
        out_shape=out_shapes,
    )(*args)


def kernel(params, x, edge_index, edge_attr, batch):
    f32 = jnp.float32
    src = edge_index[0]
    dst = edge_index[1]

    na_w = jax.nn.softmax(params["na_alpha"] / 0.001, axis=-1)
    fu_w = jax.nn.softmax(params["fu_alpha"] / 0.001, axis=-1)
    se_w = [jax.nn.softmax(a / 0.001, axis=-1)[:, 0] for a in params["se_alpha"]]
    Wn, bn_, We = params["na_W"], params["na_b"], params["na_We"]
    Wstack = [jnp.stack([na_w[i, k] * Wn[i, k] for k in range(3)], 0) for i in range(L)]
    G = [sum(na_w[i, k] * (We[i, k] @ Wn[i, k]) for k in range(3)) for i in range(L)]
    btil = [sum(na_w[i, k] * bn_[i, k] for k in range(3)) for i in range(L)]

    gamma, beta = params["bn_gamma"], params["bn_beta"]
    h0row = params["node_emb"][0]
    c = jax.nn.elu(beta[0])
    u = c @ (na_w[0, 0] * Wn[0, 0])
    v = c @ (na_w[0, 1] * Wn[0, 1] + na_w[0, 2] * Wn[0, 2])

    E, deg16, slist, dlist, cnts = _bucket(src, dst, edge_attr)

    vecs1 = jnp.stack([u, v, btil[0], h0row, gamma[1], beta[1]], 0)
    sf1 = jnp.concatenate([se_w[1], fu_w[1]]).reshape(5, 1).astype(f32)
    h1, x2_2 = _call_tc(
        _tc1,
        [jax.ShapeDtypeStruct((NPAD, HID), f32),
         jax.ShapeDtypeStruct((NPAD, HID), f32)],
        E, deg16, G[0], vecs1, sf1)

    s2, m2 = _aggregate(slist, dlist, cnts, x2_2)

    vecs2 = jnp.stack([btil[1], h0row, gamma[2], beta[2]], 0)
    sf2 = jnp.concatenate([se_w[2], fu_w[2]]).reshape(6, 1).astype(f32)
    h2, x2_3 = _call_tc(
        functools.partial(_tc_mid, 1),
        [jax.ShapeDtypeStruct((NPAD, HID), f32),
         jax.ShapeDtypeStruct((NPAD, HID), f32)],
        s2, m2, E, deg16, Wstack[1], G[1], vecs2, sf2, h1)

    s3, m3 = _aggregate(slist, dlist, cnts, x2_3)

    vecs3 = jnp.stack([btil[2], h0row, gamma[3], beta[3]], 0)
    sf3 = jnp.concatenate([se_w[3], fu_w[3]]).reshape(7, 1).astype(f32)
    _h3, x2f = _call_tc(
        functools.partial(_tc_mid, 2),
        [jax.ShapeDtypeStruct((NPAD, HID), f32),
         jax.ShapeDtypeStruct((NPAD, HID), f32)],
        s3, m3, E, deg16, Wstack[2], G[2], vecs3, sf3, h1, h2)

    batch_p = jnp.concatenate([batch, jnp.zeros((NPAD - N,), jnp.int32)])
    pp = _pool(batch_p, x2f)

    out = _call_tc(
        _tc4,
        jax.ShapeDtypeStruct((NG, ODIM), f32),
        pp, params["cls_W"], params["cls_b"].reshape(1, ODIM))
    return out


# speedup vs baseline: 1.9234x; 1.3748x over previous
import functools
import jax
import jax.numpy as jnp
from jax import lax
from jax.experimental import pallas as pl
from jax.experimental.pallas import tpu as pltpu
from jax.experimental.pallas import tpu_sc as plsc

N = 10000
NE = 320000
HID = 128
DE = 16
L = 3
NG = 128
ODIM = 10

NW = 32
NPT = 320
NPAD = NW * NPT
W = 128
CH = 256
EPT = NE // NW
WE = 2000
ROWS_PER_TILE = NPAD // 16

_mesh = plsc.VectorSubcoreMesh(core_axis_name="c", subcore_axis_name="s")


def _wid():
    return lax.axis_index("s") * 2 + lax.axis_index("c")


CAP = NE

@functools.partial(
    pl.kernel,
    mesh=_mesh,
    out_type=[
        jax.ShapeDtypeStruct((NPAD, DE), jnp.float32),
        jax.ShapeDtypeStruct((NPAD, DE), jnp.float32),
        jax.ShapeDtypeStruct((NW, CAP), jnp.int32),
        jax.ShapeDtypeStruct((NW, CAP), jnp.int32),
        jax.ShapeDtypeStruct((NW, 16), jnp.int32),
    ],
    scratch_types=[
        pltpu.VMEM((CH,), jnp.int32),
        pltpu.VMEM((CH,), jnp.int32),
        pltpu.VMEM((CH, DE), jnp.float32),
        pltpu.VMEM((W + 32,), jnp.int32),
        pltpu.VMEM((W + 32,), jnp.int32),
        pltpu.VMEM((16,), jnp.int32),
        pltpu.VMEM((48,), jnp.int32),
        pltpu.VMEM((NPT + 16, DE), jnp.float32),
        pltpu.VMEM((NPT + 16, DE), jnp.float32),
    ],
)
def _bucket(src_hbm, dst_hbm, ea_hbm, eout, dout, slist, dlist, cnts,
            dvm, svm, eav, st_src, st_dst, cntb, red, acc_e, acc_d):
    wid = _wid()
    lo = wid * NPT
    iota16 = lax.iota(jnp.int32, 16)
    zi = jnp.zeros((16,), jnp.int32)
    zv = jnp.zeros((16,), jnp.float32)
    ov = jnp.ones((16,), jnp.float32)

    def init(i, _):
        acc_e[i, pl.ds(0, DE)] = zv
        acc_d[i, pl.ds(0, DE)] = zv
        return 0
    lax.fori_loop(0, NPT + 16, init, 0)

    def flush(wc):
        pltpu.sync_copy(st_src.at[pl.ds(0, W)], slist.at[wid, pl.ds(wc * W, W)])
        pltpu.sync_copy(st_dst.at[pl.ds(0, W)], dlist.at[wid, pl.ds(wc * W, W)])
        return wc + 1

    def chunk(ci, carry):
        off = ci * CH
        pltpu.sync_copy(dst_hbm.at[pl.ds(off, CH)], dvm)
        pltpu.sync_copy(src_hbm.at[pl.ds(off, CH)], svm)
        pltpu.sync_copy(ea_hbm.at[pl.ds(off, CH)], eav)

        def vreg(k, carry):
            n, wc = carry
            d16 = dvm[pl.ds(k * 16, 16)]
            dloc = d16 - lo
            m = (dloc >= 0) & (dloc < NPT)
            mi = jnp.where(m, 1, 0)
            red[pl.ds(16, 16)] = mi
            red[pl.ds(0, 16)] = zi
            x = red[pl.ds(16, 16)] + red[pl.ds(15, 16)]
            red[pl.ds(16, 16)] = x
            x = red[pl.ds(16, 16)] + red[pl.ds(14, 16)]
            red[pl.ds(16, 16)] = x
            x = red[pl.ds(16, 16)] + red[pl.ds(12, 16)]
            red[pl.ds(16, 16)] = x
            x = red[pl.ds(16, 16)] + red[pl.ds(8, 16)]
            tot = x[15]

            def work(carry):
                n, wc = carry
                s16 = svm[pl.ds(k * 16, 16)]
                dls = jnp.where(m, dloc, NPT + (iota16 & 7))
                for l in range(16):
                    st_src[pl.ds(n, 16)] = jnp.full((16,), s16[l], jnp.int32)
                    st_dst[pl.ds(n, 16)] = jnp.full((16,), dloc[l], jnp.int32)
                    dl = dls[l]
                    acc_e[dl, pl.ds(0, DE)] = acc_e[dl, pl.ds(0, DE)] + eav[k * 16 + l, pl.ds(0, DE)]
                    acc_d[dl, pl.ds(0, DE)] = acc_d[dl, pl.ds(0, DE)] + ov
                    n = n + mi[l]

                def fl(carry):
                    n, wc = carry
                    wc = flush(wc)
                    st_src[pl.ds(0, 16)] = st_src[pl.ds(W, 16)]
                    st_dst[pl.ds(0, 16)] = st_dst[pl.ds(W, 16)]
                    return n - W, wc

                return lax.cond(n >= W, fl, lambda c: c, (n, wc))

            return lax.cond(tot > 0, work, lambda c: c, (n, wc))

        return lax.fori_loop(0, CH // 16, vreg, carry)

    n, wc = lax.fori_loop(0, NE // CH, chunk, (jnp.int32(0), jnp.int32(0)))

    def pad(k, _):
        vals_s = st_src[pl.ds(k * 16, 16)]
        vals_d = st_dst[pl.ds(k * 16, 16)]
        ok = (k * 16 + iota16) < n
        st_src[pl.ds(k * 16, 16)] = jnp.where(ok, vals_s, 0)
        st_dst[pl.ds(k * 16, 16)] = jnp.where(ok, vals_d, NPT)
        return 0
    lax.fori_loop(0, W // 16, pad, 0)
    wc = flush(wc)

    cntb[pl.ds(0, 16)] = jnp.full((16,), wc * W, jnp.int32)
    pltpu.sync_copy(cntb, cnts.at[wid])

    pltpu.sync_copy(acc_e.at[pl.ds(0, NPT)], eout.at[pl.ds(lo, NPT)])
    pltpu.sync_copy(acc_d.at[pl.ds(0, NPT)], dout.at[pl.ds(lo, NPT)])


@functools.partial(
    pl.kernel,
    mesh=_mesh,
    out_type=[
        jax.ShapeDtypeStruct((NPAD, HID), jnp.float32),
        jax.ShapeDtypeStruct((NPAD, HID), jnp.float32),
    ],
    scratch_types=[
        pltpu.VMEM((16,), jnp.int32),
        pltpu.VMEM((W,), jnp.int32),
        pltpu.VMEM((W + 16,), jnp.int32),
        pltpu.VMEM((W, HID), jnp.float32),
        pltpu.VMEM((NPT + 1, HID), jnp.float32),
        pltpu.VMEM((NPT + 1, HID), jnp.float32),
    ],
)
def _aggregate(slist, dlist, cnts, x_hbm, sum_out, max_out,
               cntv, gidx, dstw, rows_v, acc_s, acc_m):
    wid = _wid()
    lo = wid * NPT
    zv = jnp.zeros((16,), jnp.float32)
    nv = jnp.full((16,), -3.0e38, jnp.float32)

    def init(i, _):
        for j in range(8):
            acc_s[i, pl.ds(j * 16, 16)] = zv
            acc_m[i, pl.ds(j * 16, 16)] = nv
        return 0
    lax.fori_loop(0, NPT + 1, init, 0)

    pltpu.sync_copy(cnts.at[wid], cntv)
    nwin = cntv[pl.ds(0, 16)][0] // W

    def win(w, _):
        pltpu.sync_copy(slist.at[wid, pl.ds(w * W, W)], gidx)
        pltpu.sync_copy(dlist.at[wid, pl.ds(w * W, W)], dstw.at[pl.ds(0, W)])
        pltpu.sync_copy(x_hbm.at[gidx], rows_v)

        def edge(e, _):
            dl = dstw[pl.ds(e, 16)][0]
            for j in range(8):
                r = rows_v[e, pl.ds(j * 16, 16)]
                acc_s[dl, pl.ds(j * 16, 16)] = acc_s[dl, pl.ds(j * 16, 16)] + r
                acc_m[dl, pl.ds(j * 16, 16)] = jnp.maximum(acc_m[dl, pl.ds(j * 16, 16)], r)
            return 0
        lax.fori_loop(0, W, edge, 0)
        return 0

    lax.fori_loop(0, nwin, win, 0)

    pltpu.sync_copy(acc_s.at[pl.ds(0, NPT)], sum_out.at[pl.ds(lo, NPT)])
    pltpu.sync_copy(acc_m.at[pl.ds(0, NPT)], max_out.at[pl.ds(lo, NPT)])


@functools.partial(
    pl.kernel,
    mesh=_mesh,
    out_type=jax.ShapeDtypeStruct((2, NG, HID), jnp.float32),
    scratch_types=[
        pltpu.VMEM((NPT,), jnp.int32),
        pltpu.VMEM((NPT, HID), jnp.float32),
        pltpu.VMEM((8, HID), jnp.float32),
        pltpu.VMEM_SHARED((NG, HID), jnp.float32),
    ],
)
def _pool(batch_hbm, x_hbm, out, bidx, rows_v, z_v, shp):
    cid = lax.axis_index("c")
    sid = lax.axis_index("s")
    zv = jnp.zeros((16,), jnp.float32)

    def zfill(i, _):
        r = i // 8
        col = (i - r * 8) * 16
        z_v[r, pl.ds(col, 16)] = zv
        return 0
    lax.fori_loop(0, 8 * HID // 16, zfill, 0)
    pltpu.sync_copy(z_v, shp.at[pl.ds(sid * 8, 8)])
    plsc.subcore_barrier()

    lo = _wid() * NPT
    pltpu.sync_copy(batch_hbm.at[pl.ds(lo, NPT)], bidx)
    pltpu.sync_copy(x_hbm.at[pl.ds(lo, NPT)], rows_v)
    pltpu.sync_copy(rows_v, shp.at[bidx], add=True)
    plsc.subcore_barrier()

    @pl.when(sid == 0)
    def _():
        pltpu.sync_copy(shp, out.at[cid])


def _bn_elu(y, gamma, beta, mask, apply_elu):
    cnt = jnp.float32(N)
    ym = jnp.where(mask, y, 0.0)
    mean = jnp.sum(ym, axis=0, keepdims=True) / cnt
    d = y - mean
    var = jnp.sum(jnp.where(mask, d * d, 0.0), axis=0, keepdims=True) / cnt
    out = d / jnp.sqrt(var + 1e-5) * gamma + beta
    if apply_elu:
        out = jnp.where(out > 0, out, jnp.exp(jnp.minimum(out, 0.0)) - 1.0)
    return jnp.where(mask, out, 0.0)


def _rowmask():
    return lax.broadcasted_iota(jnp.int32, (NPAD, 1), 0) < N


def _tc1(e_in, d_in, g0_ref, vecs_ref, sf_ref, h1_ref, x2_ref):
    E = e_in[...]
    deg = d_in[:, 0:1]
    u = vecs_ref[0:1, :]
    v = vecs_ref[1:2, :]
    b0 = vecs_ref[2:3, :]
    h0 = vecs_ref[3:4, :]
    gam = vecs_ref[4:5, :]
    bet = vecs_ref[5:6, :]
    mask = _rowmask()
    h1 = deg * u + jnp.where(deg > 0, 1.0, 0.0) * v \
        + jnp.dot(E, g0_ref[...], preferred_element_type=jnp.float32) + b0
    se0 = sf_ref[0:1, 0:1]
    se1 = sf_ref[1:2, 0:1]
    f0 = sf_ref[2:3, 0:1]
    f1 = sf_ref[3:4, 0:1]
    f2 = sf_ref[4:5, 0:1]
    a = se0 * h0
    bmat = se1 * h1
    s = a + bmat
    y = f0 * s + f1 * (s * 0.5) + f2 * jnp.maximum(a, bmat)
    x2 = _bn_elu(y, gam, bet, mask, True)
    h1_ref[...] = h1
    x2_ref[...] = x2


def _tc_mid(nh, sum_ref, max_ref, e_ref, deg_ref, ws_ref, g_ref, vecs_ref, sf_ref,
            *rest):
    hs_refs = rest[:nh]
    h_ref, x2_ref = rest[nh], rest[nh + 1]
    deg = deg_ref[:, 0:1]
    invd = 1.0 / jnp.maximum(deg, 1.0)
    asum = sum_ref[...]
    amax = jnp.where(deg > 0, max_ref[...], 0.0)
    b_ = vecs_ref[0:1, :]
    h0 = vecs_ref[1:2, :]
    gam = vecs_ref[2:3, :]
    bet = vecs_ref[3:4, :]
    h = jnp.dot(asum, ws_ref[0], preferred_element_type=jnp.float32) \
        + jnp.dot(asum * invd, ws_ref[1], preferred_element_type=jnp.float32) \
        + jnp.dot(amax, ws_ref[2], preferred_element_type=jnp.float32) \
        + jnp.dot(e_ref[...], g_ref[...], preferred_element_type=jnp.float32) + b_
    hs = [h0] + [r[...] for r in hs_refs] + [h]
    k = len(hs)
    terms = [sf_ref[j:j + 1, 0:1] * hs[j] for j in range(k)]
    s = terms[0]
    mx = terms[0]
    for t in terms[1:]:
        s = s + t
        mx = jnp.maximum(mx, t)
    f0 = sf_ref[k:k + 1, 0:1]
    f1 = sf_ref[k + 1:k + 2, 0:1]
    f2 = sf_ref[k + 2:k + 3, 0:1]
    y = f0 * s + f1 * (s / jnp.float32(k)) + f2 * mx
    mask = _rowmask()
    x2 = _bn_elu(y, gam, bet, mask, nh < 2)
    h_ref[...] = h
    x2_ref[...] = x2


def _tc4(pp_ref, cw_ref, cb_ref, out_ref):
    pooled = pp_ref[0] + pp_ref[1]
    out_ref[...] = jnp.dot(pooled, cw_ref[...], preferred_element_type=jnp.float32) \
        + cb_ref[0:1, :]


def _call_tc(fn, out_shapes, *args):
    return pl.pallas_call(
        fn,
        out_shape=out_shapes,
    )(*args)


def kernel(params, x, edge_index, edge_attr, batch):
    f32 = jnp.float32
    src = edge_index[0]
    dst = edge_index[1]

    na_w = jax.nn.softmax(params["na_alpha"] / 0.001, axis=-1)
    fu_w = jax.nn.softmax(params["fu_alpha"] / 0.001, axis=-1)
    se_w = [jax.nn.softmax(a / 0.001, axis=-1)[:, 0] for a in params["se_alpha"]]
    Wn, bn_, We = params["na_W"], params["na_b"], params["na_We"]
    Wstack = [jnp.stack([na_w[i, k] * Wn[i, k] for k in range(3)], 0) for i in range(L)]
    G = [sum(na_w[i, k] * (We[i, k] @ Wn[i, k]) for k in range(3)) for i in range(L)]
    btil = [sum(na_w[i, k] * bn_[i, k] for k in range(3)) for i in range(L)]

    gamma, beta = params["bn_gamma"], params["bn_beta"]
    h0row = params["node_emb"][0]
    c = jax.nn.elu(beta[0])
    u = c @ (na_w[0, 0] * Wn[0, 0])
    v = c @ (na_w[0, 1] * Wn[0, 1] + na_w[0, 2] * Wn[0, 2])

    E, deg16, slist, dlist, cnts = _bucket(src, dst, edge_attr)

    vecs1 = jnp.stack([u, v, btil[0], h0row, gamma[1], beta[1]], 0)
    sf1 = jnp.concatenate([se_w[1], fu_w[1]]).reshape(5, 1).astype(f32)
    h1, x2_2 = _call_tc(
        _tc1,
        [jax.ShapeDtypeStruct((NPAD, HID), f32),
         jax.ShapeDtypeStruct((NPAD, HID), f32)],
        E, deg16, G[0], vecs1, sf1)

    s2, m2 = _aggregate(slist, dlist, cnts, x2_2)

    vecs2 = jnp.stack([btil[1], h0row, gamma[2], beta[2]], 0)
    sf2 = jnp.concatenate([se_w[2], fu_w[2]]).reshape(6, 1).astype(f32)
    h2, x2_3 = _call_tc(
        functools.partial(_tc_mid, 1),
        [jax.ShapeDtypeStruct((NPAD, HID), f32),
         jax.ShapeDtypeStruct((NPAD, HID), f32)],
        s2, m2, E, deg16, Wstack[1], G[1], vecs2, sf2, h1)

    s3, m3 = _aggregate(slist, dlist, cnts, x2_3)

    vecs3 = jnp.stack([btil[2], h0row, gamma[3], beta[3]], 0)
    sf3 = jnp.concatenate([se_w[3], fu_w[3]]).reshape(7, 1).astype(f32)
    _h3, x2f = _call_tc(
        functools.partial(_tc_mid, 2),
        [jax.ShapeDtypeStruct((NPAD, HID), f32),
         jax.ShapeDtypeStruct((NPAD, HID), f32)],
        s3, m3, E, deg16, Wstack[2], G[2], vecs3, sf3, h1, h2)

    batch_p = jnp.concatenate([batch, jnp.zeros((NPAD - N,), jnp.int32)])
    pp = _pool(batch_p, x2f)

    out = _call_tc(
        _tc4,
        jax.ShapeDtypeStruct((NG, ODIM), f32),
        pp, params["cls_W"], params["cls_b"].reshape(1, ODIM))
    return out
